# baseline (speedup 1.0000x reference)
import functools

import jax
import jax.numpy as jnp
from jax import lax
from jax.experimental import pallas as pl
from jax.experimental.pallas import tpu as pltpu
from jax.experimental.pallas import tpu_sc as plsc

NUM_HEADS = 26
N_PER_HEAD = 100000
D = 32
BATCH = 16384
TOTAL = BATCH * NUM_HEADS
NUM_ROWS = NUM_HEADS * N_PER_HEAD
NUM_WORKERS = 32
COL_GROUPS = NUM_ROWS // 128
TAIL_BASE = COL_GROUPS * 128
GROUPS_PER_W = (COL_GROUPS + NUM_WORKERS - 1) // NUM_WORKERS
UNITS_PER_W = (NUM_HEADS * (BATCH // 128)) // NUM_WORKERS
IDS_PER_W = TOTAL // NUM_WORKERS


def _detile_body(ids_hbm, tab_hbm, scr_hbm, fid_hbm,
                 in0, in1, ou0, ou1, idb,
                 is0, is1, os0, os1):
    wid = lax.axis_index("s") * 2 + lax.axis_index("c")
    lane = lax.iota(jnp.int32, 16)
    dv0 = lane
    dv1 = lane + 16

    q0 = wid * IDS_PER_W

    def idchunk(j, carry):
        q = q0 + j * 512
        pltpu.sync_copy(
            ids_hbm.at[q // BATCH, pl.ds(q % BATCH, 512)],
            idb.at[pl.ds(j * 512, 512)],
        )
        return carry

    lax.fori_loop(0, IDS_PER_W // 512, idchunk, 0)

    @plsc.parallel_loop(0, IDS_PER_W // 16, unroll=8)
    def _(i):
        off = ((q0 + i * 16) // BATCH) * N_PER_HEAD
        idb[pl.ds(i * 16, 16)] = idb[pl.ds(i * 16, 16)] + off

    pltpu.sync_copy(idb, fid_hbm.at[pl.ds(q0, IDS_PER_W)])

    n_mine = (COL_GROUPS - wid + NUM_WORKERS - 1) // NUM_WORKERS
    ins = (in0, in1)
    ous = (ou0, ou1)
    isems = (is0, is1)
    osems = (os0, os1)

    def issue_in(k, b):
        g = wid + k * NUM_WORKERS
        pltpu.async_copy(tab_hbm.at[:, pl.ds(g * 128, 128)], ins[b], isems[b])

    def wait_in(b):
        pltpu.make_async_copy(
            tab_hbm.at[:, pl.ds(0, 128)], ins[b], isems[b]
        ).wait()

    def issue_out(k, b):
        g = wid + k * NUM_WORKERS
        pltpu.async_copy(ous[b], scr_hbm.at[pl.ds(g * 128, 128), :], osems[b])

    def wait_out(b):
        pltpu.make_async_copy(
            ous[b], scr_hbm.at[pl.ds(0, 128), :], osems[b]
        ).wait()

    def transpose(b):
        src = ins[b]
        dst = ous[b]
        for c in range(128):
            cv = jnp.full((16,), c, jnp.int32)
            dst[c, pl.ds(0, 16)] = plsc.load_gather(src, [dv0, cv])
            dst[c, pl.ds(16, 16)] = plsc.load_gather(src, [dv1, cv])

    @pl.when(n_mine > 0)
    def _():
        issue_in(0, 0)

    def outer(k2, carry):
        for b in (0, 1):
            k = k2 * 2 + b

            @pl.when(k + 1 < n_mine)
            def _():
                issue_in(k + 1, 1 - b)

            @pl.when(k < n_mine)
            def _():
                wait_in(b)

                @pl.when(k >= 2)
                def _():
                    wait_out(b)

                transpose(b)
                issue_out(k, b)

        return carry

    lax.fori_loop(0, (GROUPS_PER_W + 1) // 2, outer, 0)

    @pl.when(n_mine >= 1)
    def _():
        wait_out(0)

    @pl.when(n_mine >= 2)
    def _():
        wait_out(1)


_detile = functools.partial(
    pl.kernel,
    out_type=(
        jax.ShapeDtypeStruct((NUM_ROWS, D), jnp.float32),
        jax.ShapeDtypeStruct((TOTAL,), jnp.int32),
    ),
    scratch_types=[
        pltpu.VMEM((D, 128), jnp.float32),
        pltpu.VMEM((D, 128), jnp.float32),
        pltpu.VMEM((128, D), jnp.float32),
        pltpu.VMEM((128, D), jnp.float32),
        pltpu.VMEM((IDS_PER_W,), jnp.int32),
        pltpu.SemaphoreType.DMA,
        pltpu.SemaphoreType.DMA,
        pltpu.SemaphoreType.DMA,
        pltpu.SemaphoreType.DMA,
    ],
    mesh=plsc.VectorSubcoreMesh(core_axis_name="c", subcore_axis_name="s"),
    compiler_params=pltpu.CompilerParams(needs_layout_passes=False),
)(_detile_body)


def _gather_body(fid_hbm, scr_hbm, tail_hbm, out_hbm,
                 id0, id1, rm_v, r0, r1, t0, t1,
                 gs0, gs1, js0, js1, ws0, ws1):
    wid = lax.axis_index("s") * 2 + lax.axis_index("c")
    lane = lax.iota(jnp.int32, 16)
    pltpu.sync_copy(tail_hbm, r0.at[pl.ds(128, 64)])
    pltpu.sync_copy(tail_hbm, r1.at[pl.ds(128, 64)])

    ids = (id0, id1)
    rs = (r0, r1)
    ts = (t0, t1)
    gsems = (gs0, gs1)
    jsems = (js0, js1)
    wsems = (ws0, ws1)
    u0 = wid * UNITS_PER_W

    def issue_ids(k, b):
        pltpu.async_copy(
            fid_hbm.at[pl.ds((u0 + k) * 128, 128)], ids[b], jsems[b]
        )

    def wait_ids(b):
        pltpu.make_async_copy(
            fid_hbm.at[pl.ds(0, 128)], ids[b], jsems[b]
        ).wait()

    def issue_gather(b):
        pltpu.async_copy(scr_hbm.at[ids[b]], rs[b].at[pl.ds(0, 128)], gsems[b])

    def wait_gather(b):
        pltpu.make_async_copy(
            scr_hbm.at[ids[b]], rs[b].at[pl.ds(0, 128)], gsems[b]
        ).wait()

    def issue_out(k, b):
        gu = u0 + k
        h = gu // 128
        s = gu % 128
        for q in range(4):
            pltpu.async_copy(ts[b].at[q], out_hbm.at[h, q, s], wsems[b])

    def wait_out(b):
        for q in range(4):
            pltpu.make_async_copy(
                ts[b].at[q], out_hbm.at[0, q, 0], wsems[b]
            ).wait()

    def transpose(b):
        src = rs[b]
        dst = ts[b]
        for c8 in range(8):
            c0 = c8 * 16
            rv = rm_v[pl.ds(c0, 16)]
            for d in range(32):
                dv = jnp.full((16,), d, jnp.int32)
                dst[d // 8, d % 8, pl.ds(c0, 16)] = plsc.load_gather(
                    src, [rv, dv]
                )

    pltpu.sync_copy(fid_hbm.at[pl.ds(u0 * 128, 128)], id0)
    issue_gather(0)
    issue_ids(1, 1)

    def outer(k2, carry):
        for b in (0, 1):
            k = k2 * 2 + b
            wait_gather(b)

            @pl.when(k + 1 < UNITS_PER_W)
            def _():
                wait_ids(1 - b)
                issue_gather(1 - b)

            for j in range(8):
                iv = ids[b][pl.ds(j * 16, 16)]
                rm_v[pl.ds(j * 16, 16)] = jnp.where(
                    iv >= TAIL_BASE, iv - (TAIL_BASE - 128), lane + j * 16
                )

            @pl.when(k + 2 < UNITS_PER_W)
            def _():
                issue_ids(k + 2, b)

            @pl.when(k >= 2)
            def _():
                wait_out(b)

            transpose(b)
            issue_out(k, b)

        return carry

    lax.fori_loop(0, UNITS_PER_W // 2, outer, 0)
    wait_out(0)
    wait_out(1)


_gather = functools.partial(
    pl.kernel,
    out_type=jax.ShapeDtypeStruct((NUM_HEADS, 4, 128, 8, 128), jnp.float32),
    scratch_types=[
        pltpu.VMEM((128,), jnp.int32),
        pltpu.VMEM((128,), jnp.int32),
        pltpu.VMEM((128,), jnp.int32),
        pltpu.VMEM((192, D), jnp.float32),
        pltpu.VMEM((192, D), jnp.float32),
        pltpu.VMEM((4, 8, 128), jnp.float32),
        pltpu.VMEM((4, 8, 128), jnp.float32),
        pltpu.SemaphoreType.DMA,
        pltpu.SemaphoreType.DMA,
        pltpu.SemaphoreType.DMA,
        pltpu.SemaphoreType.DMA,
        pltpu.SemaphoreType.DMA,
        pltpu.SemaphoreType.DMA,
    ],
    mesh=plsc.VectorSubcoreMesh(core_axis_name="c", subcore_axis_name="s"),
    compiler_params=pltpu.CompilerParams(
        use_tc_tiling_on_sc=False, needs_layout_passes=False
    ),
)(_gather_body)


def kernel(input_ids, table):
    ids_t = input_ids.T
    tab_t = table.T
    tail = lax.slice(table, (TAIL_BASE, 0), (NUM_ROWS, D))
    scr, fid = _detile(ids_t, tab_t)
    out5 = _gather(fid, scr, tail)
    out3 = out5.transpose(0, 1, 3, 2, 4).reshape(NUM_HEADS, D, BATCH)
    return out3.transpose(2, 0, 1)


# speedup vs baseline: 2.4306x; 2.4306x over previous
import functools

import jax
import jax.numpy as jnp
from jax import lax
from jax.experimental import pallas as pl
from jax.experimental.pallas import tpu as pltpu
from jax.experimental.pallas import tpu_sc as plsc

NUM_HEADS = 26
N_PER_HEAD = 100000
D = 32
BATCH = 16384
TOTAL = BATCH * NUM_HEADS
NUM_ROWS = NUM_HEADS * N_PER_HEAD
NUM_WORKERS = 32
COL_GROUPS = NUM_ROWS // 128
TAIL_BASE = COL_GROUPS * 128
TAIL_LINE = TAIL_BASE // 4
NUM_LINES = (NUM_ROWS + 3) // 4
GROUPS_PER_W = (COL_GROUPS + NUM_WORKERS - 1) // NUM_WORKERS
UNITS_PER_W = (NUM_HEADS * (BATCH // 128)) // NUM_WORKERS
IDS_PER_W = TOTAL // NUM_WORKERS


def _body(ids_hbm, tab_hbm, tail_hbm, out_hbm, scr_hbm,
          idb, in0, in1, ou0, ou1, i40, i41, rm_v, cb_v, r0, r1, t0, t1,
          is0, is1, os0, os1, gs0, gs1, ws0, ws1, xsem):
    cid = lax.axis_index("c")
    sid = lax.axis_index("s")
    wid = sid * 2 + cid
    lane = lax.iota(jnp.int32, 16)
    dv0 = lane
    dv1 = lane + 16

    q0 = wid * IDS_PER_W

    def idchunk(j, carry):
        q = q0 + j * 512
        pltpu.sync_copy(
            ids_hbm.at[q // BATCH, pl.ds(q % BATCH, 512)],
            idb.at[pl.ds(j * 512, 512)],
        )
        return carry

    lax.fori_loop(0, IDS_PER_W // 512, idchunk, 0)

    @plsc.parallel_loop(0, IDS_PER_W // 16, unroll=8)
    def _(i):
        off = ((q0 + i * 16) // BATCH) * N_PER_HEAD
        idb[pl.ds(i * 16, 16)] = idb[pl.ds(i * 16, 16)] + off

    n_mine = (COL_GROUPS - wid + NUM_WORKERS - 1) // NUM_WORKERS
    ins = (in0, in1)
    ous = (ou0, ou1)
    isems = (is0, is1)
    osems = (os0, os1)

    def issue_in(k, b):
        g = wid + k * NUM_WORKERS
        pltpu.async_copy(tab_hbm.at[:, pl.ds(g * 128, 128)], ins[b], isems[b])

    def wait_in(b):
        pltpu.make_async_copy(
            tab_hbm.at[:, pl.ds(0, 128)], ins[b], isems[b]
        ).wait()

    def issue_out(k, b):
        g = wid + k * NUM_WORKERS
        pltpu.async_copy(ous[b], scr_hbm.at[pl.ds(g * 32, 32), :], osems[b])

    def wait_out(b):
        pltpu.make_async_copy(
            ous[b], scr_hbm.at[pl.ds(0, 32), :], osems[b]
        ).wait()

    def transpose(b):
        src = ins[b]
        dst = ous[b]

        @plsc.parallel_loop(0, 128, unroll=8)
        def _(c):
            cv = jnp.full((16,), 0, jnp.int32) + c
            r = c // 4
            lo = (c % 4) * 32
            dst[r, pl.ds(lo, 16)] = plsc.load_gather(src, [dv0, cv])
            dst[r, pl.ds(lo + 16, 16)] = plsc.load_gather(src, [dv1, cv])

    @pl.when(n_mine > 0)
    def _():
        issue_in(0, 0)

    def outer(k2, carry):
        for b in (0, 1):
            k = k2 * 2 + b

            @pl.when(k + 1 < n_mine)
            def _():
                issue_in(k + 1, 1 - b)

            @pl.when(k < n_mine)
            def _():
                wait_in(b)

                @pl.when(k >= 2)
                def _():
                    wait_out(b)

                transpose(b)
                issue_out(k, b)

        return carry

    lax.fori_loop(0, (GROUPS_PER_W + 1) // 2, outer, 0)

    @pl.when(n_mine >= 1)
    def _():
        wait_out(0)

    @pl.when(n_mine >= 2)
    def _():
        wait_out(1)

    plsc.subcore_barrier()
    pltpu.semaphore_signal(
        xsem, 1, device_id=dict(c=1 - cid, s=sid),
        device_id_type=pltpu.DeviceIdType.MESH,
    )
    pltpu.semaphore_wait(xsem, 1)

    pltpu.sync_copy(tail_hbm, r0.at[pl.ds(128, 16)])
    pltpu.sync_copy(tail_hbm, r1.at[pl.ds(128, 16)])

    i4s = (i40, i41)
    rs = (r0, r1)
    ts = (t0, t1)
    gsems = (gs0, gs1)
    wsems = (ws0, ws1)
    u0 = wid * UNITS_PER_W

    def build_i4(k, b):
        for j in range(8):
            iv = idb[pl.ds(k * 128 + j * 16, 16)]
            i4s[b][pl.ds(j * 16, 16)] = lax.shift_right_logical(iv, 2)

    def issue_gather(b):
        pltpu.async_copy(scr_hbm.at[i4s[b]], rs[b].at[pl.ds(0, 128)], gsems[b])

    def wait_gather(b):
        pltpu.make_async_copy(
            scr_hbm.at[i4s[b]], rs[b].at[pl.ds(0, 128)], gsems[b]
        ).wait()

    def issue_outs(k, b):
        gu = u0 + k
        h = gu // 128
        s = gu % 128
        for q in range(4):
            pltpu.async_copy(ts[b].at[q], out_hbm.at[h, q, s], wsems[b])

    def wait_outs(b):
        for q in range(4):
            pltpu.make_async_copy(
                ts[b].at[q], out_hbm.at[0, q, 0], wsems[b]
            ).wait()

    def build_maps(k):
        for j in range(8):
            iv = idb[pl.ds(k * 128 + j * 16, 16)]
            i4 = lax.shift_right_logical(iv, 2)
            row = jnp.where(i4 >= TAIL_LINE, i4 - (TAIL_LINE - 128), lane + j * 16)
            rm_v[pl.ds(j * 16, 16)] = row
            cb_v[pl.ds(j * 16, 16)] = (iv & 3) * 32

    def transpose_c(b):
        src = rs[b]
        dst = ts[b]

        @plsc.parallel_loop(0, 8, unroll=8)
        def _(c8):
            c0 = c8 * 16
            rv = rm_v[pl.ds(c0, 16)]
            cb = cb_v[pl.ds(c0, 16)]
            for d in range(32):
                dst[d // 8, d % 8, pl.ds(c0, 16)] = plsc.load_gather(
                    src, [rv, cb + d]
                )

    build_i4(0, 0)
    issue_gather(0)
    build_i4(1, 1)
    issue_gather(1)

    def cunit(k2, carry):
        for b in (0, 1):
            k = k2 * 2 + b
            wait_gather(b)
            build_maps(k)

            @pl.when(k >= 2)
            def _():
                wait_outs(b)

            transpose_c(b)
            issue_outs(k, b)

            @pl.when(k + 2 < UNITS_PER_W)
            def _():
                build_i4(k + 2, b)
                issue_gather(b)

        return carry

    lax.fori_loop(0, UNITS_PER_W // 2, cunit, 0)
    wait_outs(0)
    wait_outs(1)


_mhe = functools.partial(
    pl.kernel,
    out_type=(
        jax.ShapeDtypeStruct((NUM_HEADS, 4, 128, 8, 128), jnp.float32),
        jax.ShapeDtypeStruct((NUM_LINES, 128), jnp.float32),
    ),
    scratch_types=[
        pltpu.VMEM((IDS_PER_W,), jnp.int32),
        pltpu.VMEM((D, 128), jnp.float32),
        pltpu.VMEM((D, 128), jnp.float32),
        pltpu.VMEM((D, 128), jnp.float32),
        pltpu.VMEM((D, 128), jnp.float32),
        pltpu.VMEM((128,), jnp.int32),
        pltpu.VMEM((128,), jnp.int32),
        pltpu.VMEM((128,), jnp.int32),
        pltpu.VMEM((128,), jnp.int32),
        pltpu.VMEM((144, 128), jnp.float32),
        pltpu.VMEM((144, 128), jnp.float32),
        pltpu.VMEM((4, 8, 128), jnp.float32),
        pltpu.VMEM((4, 8, 128), jnp.float32),
        pltpu.SemaphoreType.DMA,
        pltpu.SemaphoreType.DMA,
        pltpu.SemaphoreType.DMA,
        pltpu.SemaphoreType.DMA,
        pltpu.SemaphoreType.DMA,
        pltpu.SemaphoreType.DMA,
        pltpu.SemaphoreType.DMA,
        pltpu.SemaphoreType.DMA,
        pltpu.SemaphoreType.REGULAR,
    ],
    mesh=plsc.VectorSubcoreMesh(core_axis_name="c", subcore_axis_name="s"),
    compiler_params=pltpu.CompilerParams(needs_layout_passes=False),
)(_body)


def kernel(input_ids, table):
    ids_t = input_ids.T
    tab_t = table.T
    tail = lax.slice(table, (TAIL_BASE, 0), (NUM_ROWS, D)).reshape(16, 128)
    out5, _ = _mhe(ids_t, tab_t, tail)
    out3 = out5.transpose(0, 1, 3, 2, 4).reshape(NUM_HEADS, D, BATCH)
    return out3.transpose(2, 0, 1)
